# baseline (device time: 213383 ns/iter reference)
import jax
import jax.numpy as jnp
from jax import lax
from jax.experimental import pallas as pl
from jax.experimental.pallas import tpu as pltpu

T = 2048
D = 4096
V_LOCAL = 8192
TV = 256
GRID = V_LOCAL // TV


def kernel(x, W, labels):
    labels2d = labels.reshape(T, 1)

    def body(x_ref, w_ref, lab_ref, out_ref,
             lbuf_ref, acc_ref, comm_ref, send_sem, recv_sem):
        step = pl.program_id(0)
        my_x = lax.axis_index("x")
        my_y = lax.axis_index("y")
        my_z = lax.axis_index("z")
        partner = (my_x, my_y, 1 - my_z)

        @pl.when(step == 0)
        def _init():
            bar = pltpu.get_barrier_semaphore()
            pl.semaphore_signal(bar, inc=1, device_id=partner,
                                device_id_type=pl.DeviceIdType.MESH)
            pl.semaphore_wait(bar, 1)
            acc_ref[...] = jnp.zeros_like(acc_ref)
            lbuf_ref[1, :, :] = jnp.zeros((T, TV), jnp.float32)

        lbuf_ref[step % 2, :, :] = jnp.dot(
            x_ref[...], w_ref[...].astype(jnp.bfloat16),
            preferred_element_type=jnp.float32)

        def consume(jj, buf):
            lb = lbuf_ref[buf, :, :]
            s = jnp.sum(jnp.exp(lb), axis=1, dtype=jnp.float32,
                        keepdims=True)
            base = my_z * V_LOCAL + jj * TV
            col = lax.broadcasted_iota(jnp.int32, (T, TV), 1)
            hit = col == (lab_ref[...] - base)
            masked = jnp.where(hit, lb, 0.0)
            lterm = jnp.sum(masked, axis=1, dtype=jnp.float32,
                            keepdims=True)
            acc_ref[:, 0:1] += s
            acc_ref[:, 1:2] += lterm

        consume(step - 1, (step - 1) % 2)

        @pl.when(step == GRID - 1)
        def _finish():
            consume(GRID - 1, (GRID - 1) % 2)
            rdma = pltpu.make_async_remote_copy(
                src_ref=acc_ref,
                dst_ref=comm_ref,
                send_sem=send_sem,
                recv_sem=recv_sem,
                device_id=partner,
                device_id_type=pl.DeviceIdType.MESH,
            )
            rdma.start()
            rdma.wait()
            s_tot = acc_ref[:, 0:1] + comm_ref[:, 0:1] - 2.0 * TV
            l_tot = acc_ref[:, 1:2] + comm_ref[:, 1:2]
            out_ref[...] = jnp.log(s_tot) - l_tot

    out = pl.pallas_call(
        body,
        grid=(GRID,),
        out_shape=jax.ShapeDtypeStruct((T, 1), jnp.float32),
        in_specs=[
            pl.BlockSpec((T, D), lambda j: (0, 0)),
            pl.BlockSpec((D, TV), lambda j: (0, j)),
            pl.BlockSpec((T, 1), lambda j: (0, 0)),
        ],
        out_specs=pl.BlockSpec((T, 1), lambda j: (0, 0)),
        scratch_shapes=[
            pltpu.VMEM((2, T, TV), jnp.float32),
            pltpu.VMEM((T, 2), jnp.float32),
            pltpu.VMEM((T, 2), jnp.float32),
            pltpu.SemaphoreType.DMA,
            pltpu.SemaphoreType.DMA,
        ],
        compiler_params=pltpu.CompilerParams(
            collective_id=0,
            dimension_semantics=("arbitrary",),
        ),
    )(x.astype(jnp.bfloat16), W, labels2d)
    return out.reshape(T)


# device time: 197764 ns/iter; 1.0790x vs baseline; 1.0790x over previous
import jax
import jax.numpy as jnp
from jax import lax
from jax.experimental import pallas as pl
from jax.experimental.pallas import tpu as pltpu

T = 2048
D = 4096
V_LOCAL = 8192
TV = 512
GRID = V_LOCAL // TV
SUB = 2
TS = TV // SUB


def kernel(x, W, labels):
    labels2d = labels.reshape(T, 1)

    def body(x_ref, w_ref, lab_ref, out_ref,
             acc_ref, comm_ref, send_sem, recv_sem):
        step = pl.program_id(0)
        my_x = lax.axis_index("x")
        my_y = lax.axis_index("y")
        my_z = lax.axis_index("z")
        partner = (my_x, my_y, 1 - my_z)

        @pl.when(step == 0)
        def _init():
            bar = pltpu.get_barrier_semaphore()
            pl.semaphore_signal(bar, inc=1, device_id=partner,
                                device_id_type=pl.DeviceIdType.MESH)
            pl.semaphore_wait(bar, 1)
            acc_ref[...] = jnp.zeros_like(acc_ref)

        xb = x_ref[...]
        lab = lab_ref[...]
        col = lax.broadcasted_iota(jnp.int32, (T, TS), 1)

        s_parts = []
        l_parts = []
        for k in range(SUB):
            wb = w_ref[:, k * TS:(k + 1) * TS].astype(jnp.bfloat16)
            logits = jnp.dot(xb, wb, preferred_element_type=jnp.float32)
            s_parts.append(jnp.sum(jnp.exp(logits), axis=1,
                                   keepdims=True))
            base = my_z * V_LOCAL + step * TV + k * TS
            hit = col == (lab - base)
            l_parts.append(jnp.sum(jnp.where(hit, logits, 0.0), axis=1,
                                   keepdims=True))

        acc_ref[:, 0:1] += sum(s_parts)
        acc_ref[:, 1:2] += sum(l_parts)

        @pl.when(step == GRID - 1)
        def _finish():
            rdma = pltpu.make_async_remote_copy(
                src_ref=acc_ref,
                dst_ref=comm_ref,
                send_sem=send_sem,
                recv_sem=recv_sem,
                device_id=partner,
                device_id_type=pl.DeviceIdType.MESH,
            )
            rdma.start()
            rdma.wait()
            s_tot = acc_ref[:, 0:1] + comm_ref[:, 0:1]
            l_tot = acc_ref[:, 1:2] + comm_ref[:, 1:2]
            out_ref[...] = jnp.log(s_tot) - l_tot

    out = pl.pallas_call(
        body,
        grid=(GRID,),
        out_shape=jax.ShapeDtypeStruct((T, 1), jnp.float32),
        in_specs=[
            pl.BlockSpec((T, D), lambda j: (0, 0)),
            pl.BlockSpec((D, TV), lambda j: (0, j)),
            pl.BlockSpec((T, 1), lambda j: (0, 0)),
        ],
        out_specs=pl.BlockSpec((T, 1), lambda j: (0, 0)),
        scratch_shapes=[
            pltpu.VMEM((T, 2), jnp.float32),
            pltpu.VMEM((T, 2), jnp.float32),
            pltpu.SemaphoreType.DMA,
            pltpu.SemaphoreType.DMA,
        ],
        compiler_params=pltpu.CompilerParams(
            collective_id=0,
            dimension_semantics=("arbitrary",),
        ),
    )(x.astype(jnp.bfloat16), W, labels2d)
    return out.reshape(T)


# device time: 197055 ns/iter; 1.0829x vs baseline; 1.0036x over previous
import jax
import jax.numpy as jnp
from jax import lax
from jax.experimental import pallas as pl
from jax.experimental.pallas import tpu as pltpu

T = 2048
D = 4096
V_LOCAL = 8192
TV = 512
GRID = V_LOCAL // TV
SUB = 2
TS = TV // SUB


def kernel(x, W, labels):
    labels2d = labels.reshape(T, 1)

    def body(x_ref, w_ref, lab_ref, out_ref,
             acc_ref, comm_ref, send_sem, recv_sem):
        step = pl.program_id(0)
        my_x = lax.axis_index("x")
        my_y = lax.axis_index("y")
        my_z = lax.axis_index("z")
        partner = (my_x, my_y, 1 - my_z)

        @pl.when(step == 0)
        def _init():
            bar = pltpu.get_barrier_semaphore()
            pl.semaphore_signal(bar, inc=1, device_id=partner,
                                device_id_type=pl.DeviceIdType.MESH)
            pl.semaphore_wait(bar, 1)
            acc_ref[...] = jnp.zeros_like(acc_ref)

        xb = x_ref[...]
        lab = lab_ref[...]
        col = lax.broadcasted_iota(jnp.int32, (T, TS), 1)

        s_parts = []
        l_parts = []
        for k in range(SUB):
            wb = w_ref[:, k * TS:(k + 1) * TS].astype(jnp.bfloat16)
            logits = jnp.dot(xb, wb, preferred_element_type=jnp.float32)
            lb = logits.astype(jnp.bfloat16)
            s_parts.append(jnp.sum(jnp.exp(lb), axis=1,
                                   dtype=jnp.float32, keepdims=True))
            base = my_z * V_LOCAL + step * TV + k * TS
            hit = col == (lab - base)
            l_parts.append(jnp.sum(jnp.where(hit, lb, jnp.bfloat16(0.0)),
                                   axis=1, dtype=jnp.float32,
                                   keepdims=True))

        acc_ref[:, 0:1] += sum(s_parts)
        acc_ref[:, 1:2] += sum(l_parts)

        @pl.when(step == GRID - 1)
        def _finish():
            rdma = pltpu.make_async_remote_copy(
                src_ref=acc_ref,
                dst_ref=comm_ref,
                send_sem=send_sem,
                recv_sem=recv_sem,
                device_id=partner,
                device_id_type=pl.DeviceIdType.MESH,
            )
            rdma.start()
            rdma.wait()
            s_tot = acc_ref[:, 0:1] + comm_ref[:, 0:1]
            l_tot = acc_ref[:, 1:2] + comm_ref[:, 1:2]
            out_ref[...] = jnp.log(s_tot) - l_tot

    out = pl.pallas_call(
        body,
        grid=(GRID,),
        out_shape=jax.ShapeDtypeStruct((T, 1), jnp.float32),
        in_specs=[
            pl.BlockSpec((T, D), lambda j: (0, 0)),
            pl.BlockSpec((D, TV), lambda j: (0, j)),
            pl.BlockSpec((T, 1), lambda j: (0, 0)),
        ],
        out_specs=pl.BlockSpec((T, 1), lambda j: (0, 0)),
        scratch_shapes=[
            pltpu.VMEM((T, 2), jnp.float32),
            pltpu.VMEM((T, 2), jnp.float32),
            pltpu.SemaphoreType.DMA,
            pltpu.SemaphoreType.DMA,
        ],
        compiler_params=pltpu.CompilerParams(
            collective_id=0,
            dimension_semantics=("arbitrary",),
        ),
    )(x.astype(jnp.bfloat16), W, labels2d)
    return out.reshape(T)


# device time: 196795 ns/iter; 1.0843x vs baseline; 1.0013x over previous
import jax
import jax.numpy as jnp
from jax import lax
from jax.experimental import pallas as pl
from jax.experimental.pallas import tpu as pltpu

T = 2048
D = 4096
V_LOCAL = 8192
TV = 512
GRID = V_LOCAL // TV
SUB = 2
TS = TV // SUB


def kernel(x, W, labels):
    labels2d = labels.reshape(T, 1)

    def body(x_ref, w_ref, lab_ref, out_ref,
             accs_ref, accl_ref, pack_ref, comm_ref, send_sem, recv_sem):
        step = pl.program_id(0)
        my_x = lax.axis_index("x")
        my_y = lax.axis_index("y")
        my_z = lax.axis_index("z")
        partner = (my_x, my_y, 1 - my_z)

        @pl.when(step == 0)
        def _init():
            bar = pltpu.get_barrier_semaphore()
            pl.semaphore_signal(bar, inc=1, device_id=partner,
                                device_id_type=pl.DeviceIdType.MESH)
            pl.semaphore_wait(bar, 1)
            accs_ref[...] = jnp.zeros_like(accs_ref)
            accl_ref[...] = jnp.zeros_like(accl_ref)

        xb = x_ref[...]
        lab = lab_ref[...]
        col = lax.broadcasted_iota(jnp.int32, (T, TS), 1)
        ones = jnp.ones((TS, 128), jnp.bfloat16)

        s_parts = []
        l_parts = []
        for k in range(SUB):
            wb = w_ref[:, k * TS:(k + 1) * TS].astype(jnp.bfloat16)
            logits = jnp.dot(xb, wb, preferred_element_type=jnp.float32)
            lb = logits.astype(jnp.bfloat16)
            s_parts.append(jnp.dot(jnp.exp(lb), ones,
                                   preferred_element_type=jnp.float32))
            base = my_z * V_LOCAL + step * TV + k * TS
            hit = col == (lab - base)
            masked = jnp.where(hit, lb, jnp.bfloat16(0.0))
            l_parts.append(jnp.dot(masked, ones,
                                   preferred_element_type=jnp.float32))

        accs_ref[...] += sum(s_parts)
        accl_ref[...] += sum(l_parts)

        @pl.when(step == GRID - 1)
        def _finish():
            pack_ref[:, 0:1] = accs_ref[:, 0:1]
            pack_ref[:, 1:2] = accl_ref[:, 0:1]
            rdma = pltpu.make_async_remote_copy(
                src_ref=pack_ref,
                dst_ref=comm_ref,
                send_sem=send_sem,
                recv_sem=recv_sem,
                device_id=partner,
                device_id_type=pl.DeviceIdType.MESH,
            )
            rdma.start()
            rdma.wait()
            s_tot = pack_ref[:, 0:1] + comm_ref[:, 0:1]
            l_tot = pack_ref[:, 1:2] + comm_ref[:, 1:2]
            out_ref[...] = jnp.log(s_tot) - l_tot

    out = pl.pallas_call(
        body,
        grid=(GRID,),
        out_shape=jax.ShapeDtypeStruct((T, 1), jnp.float32),
        in_specs=[
            pl.BlockSpec((T, D), lambda j: (0, 0)),
            pl.BlockSpec((D, TV), lambda j: (0, j)),
            pl.BlockSpec((T, 1), lambda j: (0, 0)),
        ],
        out_specs=pl.BlockSpec((T, 1), lambda j: (0, 0)),
        scratch_shapes=[
            pltpu.VMEM((T, 128), jnp.float32),
            pltpu.VMEM((T, 128), jnp.float32),
            pltpu.VMEM((T, 2), jnp.float32),
            pltpu.VMEM((T, 2), jnp.float32),
            pltpu.SemaphoreType.DMA,
            pltpu.SemaphoreType.DMA,
        ],
        compiler_params=pltpu.CompilerParams(
            collective_id=0,
            dimension_semantics=("arbitrary",),
        ),
    )(x.astype(jnp.bfloat16), W, labels2d)
    return out.reshape(T)


# device time: 191477 ns/iter; 1.1144x vs baseline; 1.0278x over previous
import jax
import jax.numpy as jnp
from jax import lax
from jax.experimental import pallas as pl
from jax.experimental.pallas import tpu as pltpu

T = 2048
D = 4096
V_LOCAL = 8192
TV = 512
GRID = V_LOCAL // TV


def kernel(x, W, labels):
    labels2d = labels.reshape(T, 1)

    def body(x_ref, w_ref, lab_ref, out_ref,
             acc_ref, comm_ref, send_sem, recv_sem):
        step = pl.program_id(0)
        my_x = lax.axis_index("x")
        my_y = lax.axis_index("y")
        my_z = lax.axis_index("z")
        partner = (my_x, my_y, 1 - my_z)

        @pl.when(step == 0)
        def _init():
            bar = pltpu.get_barrier_semaphore()
            pl.semaphore_signal(bar, inc=1, device_id=partner,
                                device_id_type=pl.DeviceIdType.MESH)
            pl.semaphore_wait(bar, 1)
            acc_ref[...] = jnp.zeros_like(acc_ref)

        w_bf = w_ref[...].astype(jnp.bfloat16)
        logits = jnp.dot(x_ref[...], w_bf,
                         preferred_element_type=jnp.float32)

        s = jnp.sum(jnp.exp(logits), axis=1, keepdims=True)

        base = my_z * V_LOCAL + step * TV
        col = lax.broadcasted_iota(jnp.int32, (T, TV), 1) + base
        hit = col == lab_ref[...]
        lterm = jnp.sum(jnp.where(hit, logits, 0.0), axis=1, keepdims=True)

        acc_ref[:, 0:1] += s
        acc_ref[:, 1:2] += lterm

        @pl.when(step == GRID - 1)
        def _finish():
            rdma = pltpu.make_async_remote_copy(
                src_ref=acc_ref,
                dst_ref=comm_ref,
                send_sem=send_sem,
                recv_sem=recv_sem,
                device_id=partner,
                device_id_type=pl.DeviceIdType.MESH,
            )
            rdma.start()
            rdma.wait()
            s_tot = acc_ref[:, 0:1] + comm_ref[:, 0:1]
            l_tot = acc_ref[:, 1:2] + comm_ref[:, 1:2]
            out_ref[...] = jnp.log(s_tot) - l_tot

    out = pl.pallas_call(
        body,
        grid=(GRID,),
        out_shape=jax.ShapeDtypeStruct((T, 1), jnp.float32),
        in_specs=[
            pl.BlockSpec((T, D), lambda j: (0, 0)),
            pl.BlockSpec((D, TV), lambda j: (0, j)),
            pl.BlockSpec((T, 1), lambda j: (0, 0)),
        ],
        out_specs=pl.BlockSpec((T, 1), lambda j: (0, 0)),
        scratch_shapes=[
            pltpu.VMEM((T, 2), jnp.float32),
            pltpu.VMEM((T, 2), jnp.float32),
            pltpu.SemaphoreType.DMA,
            pltpu.SemaphoreType.DMA,
        ],
        compiler_params=pltpu.CompilerParams(
            collective_id=0,
            dimension_semantics=("arbitrary",),
        ),
    )(x.astype(jnp.bfloat16), W, labels2d)
    return out.reshape(T)
